# Initial kernel scaffold; baseline (speedup 1.0000x reference)
import jax
import jax.numpy as jnp
from jax.experimental import pallas as pl


def kernel(l, p):
    raise NotImplementedError("write your pallas kernel here")


# speedup vs baseline: 1258.8074x; 1258.8074x over previous
import dataclasses
import functools

import jax
import jax.numpy as jnp
import numpy as np
from jax import lax
from jax.experimental import pallas as pl
from jax.experimental.pallas import tpu as pltpu
from jax.experimental.pallas import tpu_sc as plsc

_N = 8388608
_NB = 1000
_M = 8192
_STEP = float(np.float32(1.0) / np.float32(1000.0))
_ROWS = 4096
_COLS = 2048
_CPAD = 1008
_SENTINEL = 2.0e9


def _sc_body(c_hbm, p_hbm, l_hbm, probs_hbm, t_hbm, c_v, p_v, sA, sB, sC, sD):
    pltpu.sync_copy(c_hbm, c_v)
    pltpu.sync_copy(p_hbm, p_v)

    @pl.loop(0, _M, step=16)
    def _zero(i):
        sA[pl.ds(i, 16)] = jnp.zeros((16,), jnp.int32)

    @pl.loop(0, _CPAD, step=16)
    def _hist(i):
        cv = c_v[pl.ds(i, 16)]
        x = jnp.minimum(cv * jnp.float32(_M), jnp.float32(2 * _M))
        xi = x.astype(jnp.int32)
        h = xi + (xi.astype(jnp.float32) != x).astype(jnp.int32)
        mask = h <= _M - 1
        plsc.addupdate_scatter(sA, [jnp.minimum(h, _M - 1)],
                               jnp.ones((16,), jnp.int32), mask=mask)

    def _scan_step(i, carry):
        hv = sA[pl.ds(i * 16, 16)]
        sv = jnp.cumsum(hv) + carry - 1
        sA[pl.ds(i * 16, 16)] = sv
        sC[pl.ds(i * 16, 16)] = plsc.load_gather(c_v, [sv])
        sB[pl.ds(i * 16, 16)] = plsc.load_gather(c_v, [sv + 1])
        sD[pl.ds(i * 16, 16)] = plsc.load_gather(c_v, [sv + 2])
        return carry + jnp.sum(hv)

    lax.fori_loop(0, _M // 16, _scan_step, jnp.int32(0))

    def compute(l_v, pr_v, t_v):
        @pl.loop(0, _COLS, step=16)
        def _elem(i):
            u = l_v[0, pl.ds(i, 16)]
            k = (u * jnp.float32(_M)).astype(jnp.int32)
            j0 = plsc.load_gather(sA, [k])
            cn = plsc.load_gather(sB, [k])
            c0 = plsc.load_gather(sC, [k])
            c2 = plsc.load_gather(sD, [k])
            up = u >= cn
            j = j0 + up.astype(jnp.int32)
            cl = jnp.where(up, cn, c0)
            cr = jnp.where(up, c2, cn)
            jf = j.astype(jnp.float32)
            bej = jf * jnp.float32(_STEP)
            bej1 = (jf + jnp.float32(1.0)) * jnp.float32(_STEP)
            t = bej + ((u - cl) / (cr - cl)) * (bej1 - bej)
            j2 = (t / jnp.float32(_STEP)).astype(jnp.int32)
            j2 = jnp.clip(j2, 0, _NB - 1)
            pr = plsc.load_gather(p_v, [j2])
            pr_v[0, pl.ds(i, 16)] = pr / jnp.float32(_STEP)
            t_v[0, pl.ds(i, 16)] = t

    blk = lambda: pl.BlockSpec((1, _COLS), lambda g: (g, 0))
    pltpu.emit_pipeline(
        compute,
        grid=(_ROWS,),
        in_specs=[blk()],
        out_specs=[blk(), blk()],
        core_axis_name=("c", "s"),
        dimension_semantics=(pltpu.PARALLEL,),
    )(l_hbm, probs_hbm, t_hbm)


@jax.jit
def kernel(l, p):
    total = jnp.sum(p)
    c = jnp.cumsum(p) / total
    c_full = jnp.concatenate([
        jnp.zeros((1,), jnp.float32), c,
        jnp.full((_CPAD - _NB - 1,), _SENTINEL, jnp.float32),
    ])
    p_over = jnp.concatenate([p / total, jnp.zeros((_CPAD - _NB,), jnp.float32)])
    l2 = l.reshape(_ROWS, _COLS)

    mesh = plsc.VectorSubcoreMesh(core_axis_name="c", subcore_axis_name="s")
    out = jax.ShapeDtypeStruct((_ROWS, _COLS), jnp.float32)
    cp = pltpu.CompilerParams()
    if "needs_layout_passes" in pltpu.CompilerParams.__dataclass_fields__:
        cp = dataclasses.replace(cp, needs_layout_passes=False)
    run = pl.kernel(
        _sc_body,
        out_type=(out, out),
        mesh=mesh,
        compiler_params=cp,
        scratch_types=[
            pltpu.VMEM((_CPAD,), jnp.float32),
            pltpu.VMEM((_CPAD,), jnp.float32),
            pltpu.VMEM((_M,), jnp.int32),
            pltpu.VMEM((_M,), jnp.float32),
            pltpu.VMEM((_M,), jnp.float32),
            pltpu.VMEM((_M,), jnp.float32),
        ],
    )
    probs2, t2 = run(c_full, p_over, l2)
    return probs2.reshape(_N), t2.reshape(_N)
